# baseline (speedup 1.0000x reference)
import functools

import jax
import jax.numpy as jnp
from jax import lax
from jax.experimental import pallas as pl
from jax.experimental.pallas import tpu as pltpu
from jax.experimental.pallas import tpu_sc as plsc

N_NODES = 10000
N_EDGES = 320000
D_IN = 128
D_HID = 16

NC = 2
NS = 16
NW = NC * NS

NP = 10240
DUMMY = 10016
CHUNK = 128
E_TILE = 10240
E_PAD = NW * E_TILE
Q = 8
QE = E_TILE // Q
NBUF = 4
ROWS_PER_SUB = NP // NS

BLK = 1024
GRID = NP // BLK

_mesh = plsc.VectorSubcoreMesh(core_axis_name="c", subcore_axis_name="s")


@functools.partial(
    pl.kernel,
    mesh=_mesh,
    compiler_params=pltpu.CompilerParams(use_tc_tiling_on_sc=False),
    out_type=jax.ShapeDtypeStruct((NC, NP), jnp.float32),
    scratch_types=[
        pltpu.VMEM((E_TILE,), jnp.int32),
        pltpu.VMEM((E_TILE,), jnp.float32),
        pltpu.VMEM((ROWS_PER_SUB,), jnp.float32),
        pltpu.VMEM_SHARED((NP,), jnp.float32),
    ],
)
def _sc_degree(dst_hbm, out_hbm, didx, ones, zbuf, acc):
    c = lax.axis_index("c")
    s = lax.axis_index("s")
    wid = c * NS + s
    zero16 = jnp.zeros((16,), jnp.float32)
    one16 = jnp.ones((16,), jnp.float32)

    def _fill(i, _):
        zbuf[pl.ds(i * 16, 16)] = zero16
        return _
    lax.fori_loop(0, ROWS_PER_SUB // 16, _fill, None)

    def _fill1(i, _):
        ones[pl.ds(i * 16, 16)] = one16
        return _
    lax.fori_loop(0, E_TILE // 16, _fill1, None)

    pltpu.sync_copy(zbuf, acc.at[pl.ds(s * ROWS_PER_SUB, ROWS_PER_SUB)])
    pltpu.sync_copy(dst_hbm.at[pl.ds(wid * E_TILE, E_TILE)], didx)
    plsc.subcore_barrier()

    pltpu.sync_copy(ones, acc.at[didx], add=True)
    plsc.subcore_barrier()

    pltpu.sync_copy(acc.at[pl.ds(s * ROWS_PER_SUB, ROWS_PER_SUB)],
                    out_hbm.at[c, pl.ds(s * ROWS_PER_SUB, ROWS_PER_SUB)])


@functools.partial(
    pl.kernel,
    mesh=_mesh,
    compiler_params=pltpu.CompilerParams(use_tc_tiling_on_sc=False),
    out_type=jax.ShapeDtypeStruct((NC, NP, D_HID), jnp.float32),
    scratch_types=[
        pltpu.VMEM((Q, QE), jnp.int32),
        pltpu.VMEM((Q, QE), jnp.int32),
        pltpu.VMEM((NBUF, QE, D_HID), jnp.float32),
        pltpu.VMEM((CHUNK, D_HID), jnp.float32),
        pltpu.VMEM_SHARED((NP, D_HID), jnp.float32),
        pltpu.SemaphoreType.DMA,
        [pltpu.SemaphoreType.DMA] * NBUF,
        [pltpu.SemaphoreType.DMA] * NBUF,
    ],
)
def _sc_aggregate(src_hbm, dst_hbm, tab_hbm, out_hbm,
                  sidx, didx, rows, zbuf, acc, isem, gsems, ssems):
    c = lax.axis_index("c")
    s = lax.axis_index("s")
    wid = c * NS + s
    zero16 = jnp.zeros((16,), jnp.float32)

    iload_s = pltpu.async_copy(src_hbm.at[pl.ds(wid * Q, Q), :], sidx, isem)

    def _zero(i, _):
        zbuf[i, :] = zero16
        return _
    lax.fori_loop(0, CHUNK, _zero, None)

    def _clear(k, _):
        pltpu.sync_copy(zbuf, acc.at[pl.ds(s * ROWS_PER_SUB + k * CHUNK, CHUNK), :])
        return _
    lax.fori_loop(0, ROWS_PER_SUB // CHUNK, _clear, None)

    iload_s.wait()
    pltpu.sync_copy(dst_hbm.at[pl.ds(wid * Q, Q), :], didx)
    plsc.subcore_barrier()

    gathers = [None] * Q
    scats = [None] * Q
    for q in range(min(3, Q)):
        gathers[q] = pltpu.async_copy(
            tab_hbm.at[sidx.at[q]], rows.at[q % NBUF], gsems[q % NBUF])
    for q in range(Q):
        b = q % NBUF
        gathers[q].wait()
        if q + 3 < Q:
            nb = (q + 3) % NBUF
            if scats[q - 1] is not None:
                scats[q - 1].wait()
            gathers[q + 3] = pltpu.async_copy(
                tab_hbm.at[sidx.at[q + 3]], rows.at[nb], gsems[nb])
        scats[q] = pltpu.async_copy(rows.at[b], acc.at[didx.at[q]],
                                    ssems[b], add=True)
    for q in range(max(0, Q - 4), Q):
        scats[q].wait()
    plsc.subcore_barrier()

    pltpu.sync_copy(acc.at[pl.ds(s * ROWS_PER_SUB, ROWS_PER_SUB), :],
                    out_hbm.at[c, pl.ds(s * ROWS_PER_SUB, ROWS_PER_SUB), :])


def _tc_mm1_body(x_ref, w1_ref, h_ref):
    h_ref[:, :] = jnp.dot(x_ref[:, :], w1_ref[:, :],
                          preferred_element_type=jnp.float32)


def _tc_scale_body(h_ref, degp_ref, hp_ref):
    deg = jnp.sum(degp_ref[:, :], axis=0) + 1.0
    dinv = lax.rsqrt(deg)
    hp_ref[:, :] = h_ref[:, :] * dinv[:, None]


def _tc2_body(s_ref, hp_ref, degp_ref, w2_ref, b1_ref, h2p_ref):
    deg = jnp.sum(degp_ref[:, :], axis=0) + 1.0
    dinv = lax.rsqrt(deg)
    tot = s_ref[0, :, :] + s_ref[1, :, :] + hp_ref[:, :]
    z = jnp.maximum(tot * dinv[:, None] + b1_ref[0, :], 0.0)
    h2 = jnp.dot(z, w2_ref[:, :], preferred_element_type=jnp.float32)
    h2p_ref[:, :] = h2 * dinv[:, None]


def _tc3_body(s_ref, hp_ref, degp_ref, b2_ref, out_ref):
    deg = jnp.sum(degp_ref[:, :], axis=0) + 1.0
    dinv = lax.rsqrt(deg)
    tot = s_ref[0, :, :] + s_ref[1, :, :] + hp_ref[:, :]
    out_ref[:, :] = tot * dinv[:, None] + b2_ref[0, :]


def _tc_mm1(x_p, W1):
    return pl.pallas_call(
        _tc_mm1_body,
        grid=(GRID,),
        in_specs=[
            pl.BlockSpec((BLK, D_IN), lambda i: (i, 0)),
            pl.BlockSpec((D_IN, D_HID), lambda i: (0, 0)),
        ],
        out_specs=pl.BlockSpec((BLK, D_HID), lambda i: (i, 0)),
        out_shape=jax.ShapeDtypeStruct((NP, D_HID), jnp.float32),
    )(x_p, W1)


def _tc_scale(h, degp):
    return pl.pallas_call(
        _tc_scale_body,
        grid=(GRID,),
        in_specs=[
            pl.BlockSpec((BLK, D_HID), lambda i: (i, 0)),
            pl.BlockSpec((NC, BLK), lambda i: (0, i)),
        ],
        out_specs=pl.BlockSpec((BLK, D_HID), lambda i: (i, 0)),
        out_shape=jax.ShapeDtypeStruct((NP, D_HID), jnp.float32),
    )(h, degp)


def _tc2(S, hp, degp, W2, b1):
    return pl.pallas_call(
        _tc2_body,
        grid=(GRID,),
        in_specs=[
            pl.BlockSpec((NC, BLK, D_HID), lambda i: (0, i, 0)),
            pl.BlockSpec((BLK, D_HID), lambda i: (i, 0)),
            pl.BlockSpec((NC, BLK), lambda i: (0, i)),
            pl.BlockSpec((D_HID, D_HID), lambda i: (0, 0)),
            pl.BlockSpec((1, D_HID), lambda i: (0, 0)),
        ],
        out_specs=pl.BlockSpec((BLK, D_HID), lambda i: (i, 0)),
        out_shape=jax.ShapeDtypeStruct((NP, D_HID), jnp.float32),
    )(S, hp, degp, W2, b1)


def _tc3(S, hp, degp, b2):
    return pl.pallas_call(
        _tc3_body,
        grid=(GRID,),
        in_specs=[
            pl.BlockSpec((NC, BLK, D_HID), lambda i: (0, i, 0)),
            pl.BlockSpec((BLK, D_HID), lambda i: (i, 0)),
            pl.BlockSpec((NC, BLK), lambda i: (0, i)),
            pl.BlockSpec((1, D_HID), lambda i: (0, 0)),
        ],
        out_specs=pl.BlockSpec((BLK, D_HID), lambda i: (i, 0)),
        out_shape=jax.ShapeDtypeStruct((NP, D_HID), jnp.float32),
    )(S, hp, degp, b2)


def kernel(x, W1, b1, W2, b2, edge_index):
    src = edge_index[0].astype(jnp.int32)
    dst = edge_index[1].astype(jnp.int32)
    pad = E_PAD - N_EDGES
    src_p = jnp.concatenate([src, jnp.zeros((pad,), jnp.int32)])
    dst_p = jnp.concatenate([dst, jnp.full((pad,), DUMMY, jnp.int32)])
    x_p = jnp.pad(x, ((0, NP - N_NODES), (0, 0)))
    b1r = b1.reshape(1, D_HID)
    b2r = b2.reshape(1, D_HID)

    src_q = src_p.reshape(NW * Q, QE)
    dst_q = dst_p.reshape(NW * Q, QE)

    degp = _sc_degree(dst_p)
    h1 = _tc_mm1(x_p, W1)
    h1p = _tc_scale(h1, degp)
    S1 = _sc_aggregate(src_q, dst_q, h1p)
    h2p = _tc2(S1, h1p, degp, W2, b1r)
    S2 = _sc_aggregate(src_q, dst_q, h2p)
    out = _tc3(S2, h2p, degp, b2r)
    return out[:N_NODES]


# speedup vs baseline: 1.0891x; 1.0891x over previous
import functools

import jax
import jax.numpy as jnp
from jax import lax
from jax.experimental import pallas as pl
from jax.experimental.pallas import tpu as pltpu
from jax.experimental.pallas import tpu_sc as plsc

N_NODES = 10000
N_EDGES = 320000
D_IN = 128
D_HID = 16

NC = 2
NS = 16
NW = NC * NS

NP = 10240
DUMMY = 10016
CHUNK = 128
E_TILE = 10240
E_PAD = NW * E_TILE
Q = 8
QE = E_TILE // Q
NBUF = 4
ROWS_PER_SUB = NP // NS

BLK = 1024
GRID = NP // BLK

_mesh = plsc.VectorSubcoreMesh(core_axis_name="c", subcore_axis_name="s")


@functools.partial(
    pl.kernel,
    mesh=_mesh,
    compiler_params=pltpu.CompilerParams(use_tc_tiling_on_sc=False),
    out_type=jax.ShapeDtypeStruct((NC, NP), jnp.float32),
    scratch_types=[
        pltpu.VMEM((E_TILE,), jnp.int32),
        pltpu.VMEM((E_TILE,), jnp.float32),
        pltpu.VMEM((ROWS_PER_SUB,), jnp.float32),
        pltpu.VMEM_SHARED((NP,), jnp.float32),
    ],
)
def _sc_degree(dst_hbm, out_hbm, didx, ones, zbuf, acc):
    c = lax.axis_index("c")
    s = lax.axis_index("s")
    wid = c * NS + s
    zero16 = jnp.zeros((16,), jnp.float32)
    one16 = jnp.ones((16,), jnp.float32)

    def _fill(i, _):
        zbuf[pl.ds(i * 16, 16)] = zero16
        return _
    lax.fori_loop(0, ROWS_PER_SUB // 16, _fill, None)

    def _fill1(i, _):
        ones[pl.ds(i * 16, 16)] = one16
        return _
    lax.fori_loop(0, E_TILE // 16, _fill1, None)

    pltpu.sync_copy(zbuf, acc.at[pl.ds(s * ROWS_PER_SUB, ROWS_PER_SUB)])
    pltpu.sync_copy(dst_hbm.at[pl.ds(wid * E_TILE, E_TILE)], didx)
    plsc.subcore_barrier()

    pltpu.sync_copy(ones, acc.at[didx], add=True)
    plsc.subcore_barrier()

    pltpu.sync_copy(acc.at[pl.ds(s * ROWS_PER_SUB, ROWS_PER_SUB)],
                    out_hbm.at[c, pl.ds(s * ROWS_PER_SUB, ROWS_PER_SUB)])


@functools.partial(
    pl.kernel,
    mesh=_mesh,
    compiler_params=pltpu.CompilerParams(use_tc_tiling_on_sc=False),
    out_type=jax.ShapeDtypeStruct((NC, NP, D_HID), jnp.float32),
    scratch_types=[
        pltpu.VMEM((Q, QE), jnp.int32),
        pltpu.VMEM((Q, QE), jnp.int32),
        pltpu.VMEM((NBUF, QE, D_HID), jnp.float32),
        pltpu.VMEM((CHUNK, D_HID), jnp.float32),
        pltpu.VMEM_SHARED((NP, D_HID), jnp.float32),
        pltpu.SemaphoreType.DMA,
        [pltpu.SemaphoreType.DMA] * NBUF,
        [pltpu.SemaphoreType.DMA] * NBUF,
    ],
)
def _sc_aggregate(src_hbm, dst_hbm, tab_hbm, out_hbm,
                  sidx, didx, rows, zbuf, acc, isem, gsems, ssems):
    c = lax.axis_index("c")
    s = lax.axis_index("s")
    wid = c * NS + s
    zero16 = jnp.zeros((16,), jnp.float32)

    iload_s = pltpu.async_copy(src_hbm.at[pl.ds(wid * Q, Q), :], sidx, isem)

    def _zero(i, _):
        zbuf[i, :] = zero16
        return _
    lax.fori_loop(0, CHUNK, _zero, None)

    def _clear(k, _):
        pltpu.sync_copy(zbuf, acc.at[pl.ds(s * ROWS_PER_SUB + k * CHUNK, CHUNK), :])
        return _
    lax.fori_loop(0, ROWS_PER_SUB // CHUNK, _clear, None)

    iload_s.wait()
    pltpu.sync_copy(dst_hbm.at[pl.ds(wid * Q, Q), :], didx)
    plsc.subcore_barrier()

    gathers = [None] * Q
    scats = [None] * Q
    for q in range(min(3, Q)):
        gathers[q] = pltpu.async_copy(
            tab_hbm.at[sidx.at[q]], rows.at[q % NBUF], gsems[q % NBUF])
    for q in range(Q):
        b = q % NBUF
        gathers[q].wait()
        if q + 3 < Q:
            nb = (q + 3) % NBUF
            if scats[q - 1] is not None:
                scats[q - 1].wait()
            gathers[q + 3] = pltpu.async_copy(
                tab_hbm.at[sidx.at[q + 3]], rows.at[nb], gsems[nb])
        scats[q] = pltpu.async_copy(rows.at[b], acc.at[didx.at[q]],
                                    ssems[b], add=True)
    for q in range(max(0, Q - 4), Q):
        scats[q].wait()
    plsc.subcore_barrier()

    pltpu.sync_copy(acc.at[pl.ds(s * ROWS_PER_SUB, ROWS_PER_SUB), :],
                    out_hbm.at[c, pl.ds(s * ROWS_PER_SUB, ROWS_PER_SUB), :])


def _tc1_body(x_ref, w1_ref, degp_ref, hp_ref):
    deg = jnp.sum(degp_ref[:, :], axis=0) + 1.0
    dinv = lax.rsqrt(deg)
    h = jnp.dot(x_ref[:, :], w1_ref[:, :], preferred_element_type=jnp.float32)
    hp_ref[:, :] = h * dinv[:, None]


def _tc2_body(s_ref, hp_ref, degp_ref, w2_ref, b1_ref, h2p_ref):
    deg = jnp.sum(degp_ref[:, :], axis=0) + 1.0
    dinv = lax.rsqrt(deg)
    tot = s_ref[0, :, :] + s_ref[1, :, :] + hp_ref[:, :]
    z = jnp.maximum(tot * dinv[:, None] + b1_ref[0, :], 0.0)
    h2 = jnp.dot(z, w2_ref[:, :], preferred_element_type=jnp.float32)
    h2p_ref[:, :] = h2 * dinv[:, None]


def _tc3_body(s_ref, hp_ref, degp_ref, b2_ref, out_ref):
    deg = jnp.sum(degp_ref[:, :], axis=0) + 1.0
    dinv = lax.rsqrt(deg)
    tot = s_ref[0, :, :] + s_ref[1, :, :] + hp_ref[:, :]
    out_ref[:, :] = tot * dinv[:, None] + b2_ref[0, :]


def _tc1(x_p, W1, degp):
    return pl.pallas_call(
        _tc1_body,
        grid=(GRID,),
        in_specs=[
            pl.BlockSpec((BLK, D_IN), lambda i: (i, 0)),
            pl.BlockSpec((D_IN, D_HID), lambda i: (0, 0)),
            pl.BlockSpec((NC, BLK), lambda i: (0, i)),
        ],
        out_specs=pl.BlockSpec((BLK, D_HID), lambda i: (i, 0)),
        out_shape=jax.ShapeDtypeStruct((NP, D_HID), jnp.float32),
    )(x_p, W1, degp)


def _tc2(S, hp, degp, W2, b1):
    return pl.pallas_call(
        _tc2_body,
        grid=(GRID,),
        in_specs=[
            pl.BlockSpec((NC, BLK, D_HID), lambda i: (0, i, 0)),
            pl.BlockSpec((BLK, D_HID), lambda i: (i, 0)),
            pl.BlockSpec((NC, BLK), lambda i: (0, i)),
            pl.BlockSpec((D_HID, D_HID), lambda i: (0, 0)),
            pl.BlockSpec((1, D_HID), lambda i: (0, 0)),
        ],
        out_specs=pl.BlockSpec((BLK, D_HID), lambda i: (i, 0)),
        out_shape=jax.ShapeDtypeStruct((NP, D_HID), jnp.float32),
    )(S, hp, degp, W2, b1)


def _tc3(S, hp, degp, b2):
    return pl.pallas_call(
        _tc3_body,
        grid=(GRID,),
        in_specs=[
            pl.BlockSpec((NC, BLK, D_HID), lambda i: (0, i, 0)),
            pl.BlockSpec((BLK, D_HID), lambda i: (i, 0)),
            pl.BlockSpec((NC, BLK), lambda i: (0, i)),
            pl.BlockSpec((1, D_HID), lambda i: (0, 0)),
        ],
        out_specs=pl.BlockSpec((BLK, D_HID), lambda i: (i, 0)),
        out_shape=jax.ShapeDtypeStruct((NP, D_HID), jnp.float32),
    )(S, hp, degp, b2)


def kernel(x, W1, b1, W2, b2, edge_index):
    src = edge_index[0].astype(jnp.int32)
    dst = edge_index[1].astype(jnp.int32)
    pad = E_PAD - N_EDGES
    src_p = jnp.concatenate([src, jnp.zeros((pad,), jnp.int32)])
    dst_p = jnp.concatenate([dst, jnp.full((pad,), DUMMY, jnp.int32)])
    x_p = jnp.pad(x, ((0, NP - N_NODES), (0, 0)))
    b1r = b1.reshape(1, D_HID)
    b2r = b2.reshape(1, D_HID)

    src_q = src_p.reshape(NW * Q, QE)
    dst_q = dst_p.reshape(NW * Q, QE)

    degp = _sc_degree(dst_p)
    h1p = _tc1(x_p, W1, degp)
    S1 = _sc_aggregate(src_q, dst_q, h1p)
    h2p = _tc2(S1, h1p, degp, W2, b1r)
    S2 = _sc_aggregate(src_q, dst_q, h2p)
    out = _tc3(S2, h2p, degp, b2r)
    return out[:N_NODES]


# speedup vs baseline: 1.1066x; 1.0160x over previous
import functools

import jax
import jax.numpy as jnp
from jax import lax
from jax.experimental import pallas as pl
from jax.experimental.pallas import tpu as pltpu
from jax.experimental.pallas import tpu_sc as plsc

N_NODES = 10000
N_EDGES = 320000
D_IN = 128
D_HID = 16

NC = 2
NS = 16
NW = NC * NS

NP = 10240
DUMMY = 10016
CHUNK = 128
E_TILE = 10240
E_PAD = NW * E_TILE
QE = 1280
NBUF = 4
Q0 = 12
Q1 = 4
E0_T = Q0 * QE
E1_T = Q1 * QE
E0 = NS * E0_T
ROWS_PER_SUB = NP // NS

BLK = 1024
GRID = NP // BLK

_mesh = plsc.VectorSubcoreMesh(core_axis_name="c", subcore_axis_name="s")


@functools.partial(
    pl.kernel,
    mesh=_mesh,
    compiler_params=pltpu.CompilerParams(use_tc_tiling_on_sc=False),
    out_type=jax.ShapeDtypeStruct((NC, NP), jnp.float32),
    scratch_types=[
        pltpu.VMEM((E0_T,), jnp.int32),
        pltpu.VMEM((E1_T,), jnp.int32),
        pltpu.VMEM((E0_T,), jnp.float32),
        pltpu.VMEM((ROWS_PER_SUB,), jnp.float32),
        pltpu.VMEM_SHARED((NP,), jnp.float32),
    ],
)
def _sc_degree(dst_hbm, out_hbm, didx0, didx1, ones, zbuf, acc):
    c = lax.axis_index("c")
    s = lax.axis_index("s")
    zero16 = jnp.zeros((16,), jnp.float32)
    one16 = jnp.ones((16,), jnp.float32)

    def _fill(i, _):
        zbuf[pl.ds(i * 16, 16)] = zero16
        return _
    lax.fori_loop(0, ROWS_PER_SUB // 16, _fill, None)

    def _fill1(i, _):
        ones[pl.ds(i * 16, 16)] = one16
        return _
    lax.fori_loop(0, E0_T // 16, _fill1, None)

    pltpu.sync_copy(zbuf, acc.at[pl.ds(s * ROWS_PER_SUB, ROWS_PER_SUB)])

    @pl.when(c == 0)
    def _():
        pltpu.sync_copy(dst_hbm.at[pl.ds(s * E0_T, E0_T)], didx0)

    @pl.when(c == 1)
    def _():
        pltpu.sync_copy(dst_hbm.at[pl.ds(E0 + s * E1_T, E1_T)], didx1)

    plsc.subcore_barrier()

    @pl.when(c == 0)
    def _():
        pltpu.sync_copy(ones, acc.at[didx0], add=True)

    @pl.when(c == 1)
    def _():
        pltpu.sync_copy(ones.at[pl.ds(0, E1_T)], acc.at[didx1], add=True)

    plsc.subcore_barrier()

    pltpu.sync_copy(acc.at[pl.ds(s * ROWS_PER_SUB, ROWS_PER_SUB)],
                    out_hbm.at[c, pl.ds(s * ROWS_PER_SUB, ROWS_PER_SUB)])


@functools.partial(
    pl.kernel,
    mesh=_mesh,
    compiler_params=pltpu.CompilerParams(use_tc_tiling_on_sc=False),
    out_type=jax.ShapeDtypeStruct((NC, NP, D_HID), jnp.float32),
    scratch_types=[
        pltpu.VMEM((Q0, QE), jnp.int32),
        pltpu.VMEM((Q0, QE), jnp.int32),
        pltpu.VMEM((NBUF, QE, D_HID), jnp.float32),
        pltpu.VMEM((CHUNK, D_HID), jnp.float32),
        pltpu.VMEM_SHARED((NP, D_HID), jnp.float32),
        [pltpu.SemaphoreType.DMA] * NBUF,
        [pltpu.SemaphoreType.DMA] * NBUF,
    ],
)
def _sc_aggregate(src_hbm, dst_hbm, tab_hbm, out_hbm,
                  sidx, didx, rows, zbuf, acc, gsems, ssems):
    c = lax.axis_index("c")
    s = lax.axis_index("s")
    zero16 = jnp.zeros((16,), jnp.float32)

    def _zero(i, _):
        zbuf[i, :] = zero16
        return _
    lax.fori_loop(0, CHUNK, _zero, None)

    def _clear(k, _):
        pltpu.sync_copy(zbuf, acc.at[pl.ds(s * ROWS_PER_SUB + k * CHUNK, CHUNK), :])
        return _
    lax.fori_loop(0, ROWS_PER_SUB // CHUNK, _clear, None)

    def _pipe(nq, row0):
        pltpu.sync_copy(src_hbm.at[pl.ds(row0, nq), :], sidx.at[pl.ds(0, nq), :])
        pltpu.sync_copy(dst_hbm.at[pl.ds(row0, nq), :], didx.at[pl.ds(0, nq), :])
        gathers = [None] * nq
        scats = [None] * nq
        for q in range(min(3, nq)):
            gathers[q] = pltpu.async_copy(
                tab_hbm.at[sidx.at[q]], rows.at[q % NBUF], gsems[q % NBUF])
        for q in range(nq):
            b = q % NBUF
            gathers[q].wait()
            if q + 3 < nq:
                nb = (q + 3) % NBUF
                if q - 1 >= 0:
                    scats[q - 1].wait()
                gathers[q + 3] = pltpu.async_copy(
                    tab_hbm.at[sidx.at[q + 3]], rows.at[nb], gsems[nb])
            scats[q] = pltpu.async_copy(rows.at[b], acc.at[didx.at[q]],
                                        ssems[b], add=True)
        for q in range(max(0, nq - 4), nq):
            scats[q].wait()

    plsc.subcore_barrier()

    @pl.when(c == 0)
    def _():
        _pipe(Q0, s * Q0)

    @pl.when(c == 1)
    def _():
        _pipe(Q1, NS * Q0 + s * Q1)

    plsc.subcore_barrier()

    pltpu.sync_copy(acc.at[pl.ds(s * ROWS_PER_SUB, ROWS_PER_SUB), :],
                    out_hbm.at[c, pl.ds(s * ROWS_PER_SUB, ROWS_PER_SUB), :])


def _tc1_body(x_ref, w1_ref, degp_ref, hp_ref):
    deg = jnp.sum(degp_ref[:, :], axis=0) + 1.0
    dinv = lax.rsqrt(deg)
    h = jnp.dot(x_ref[:, :], w1_ref[:, :], preferred_element_type=jnp.float32)
    hp_ref[:, :] = h * dinv[:, None]


def _tc2_body(s_ref, hp_ref, degp_ref, w2_ref, b1_ref, h2p_ref):
    deg = jnp.sum(degp_ref[:, :], axis=0) + 1.0
    dinv = lax.rsqrt(deg)
    tot = s_ref[0, :, :] + s_ref[1, :, :] + hp_ref[:, :]
    z = jnp.maximum(tot * dinv[:, None] + b1_ref[0, :], 0.0)
    h2 = jnp.dot(z, w2_ref[:, :], preferred_element_type=jnp.float32)
    h2p_ref[:, :] = h2 * dinv[:, None]


def _tc3_body(s_ref, hp_ref, degp_ref, b2_ref, out_ref):
    deg = jnp.sum(degp_ref[:, :], axis=0) + 1.0
    dinv = lax.rsqrt(deg)
    tot = s_ref[0, :, :] + s_ref[1, :, :] + hp_ref[:, :]
    out_ref[:, :] = tot * dinv[:, None] + b2_ref[0, :]


def _tc1(x_p, W1, degp):
    return pl.pallas_call(
        _tc1_body,
        grid=(GRID,),
        in_specs=[
            pl.BlockSpec((BLK, D_IN), lambda i: (i, 0)),
            pl.BlockSpec((D_IN, D_HID), lambda i: (0, 0)),
            pl.BlockSpec((NC, BLK), lambda i: (0, i)),
        ],
        out_specs=pl.BlockSpec((BLK, D_HID), lambda i: (i, 0)),
        out_shape=jax.ShapeDtypeStruct((NP, D_HID), jnp.float32),
    )(x_p, W1, degp)


def _tc2(S, hp, degp, W2, b1):
    return pl.pallas_call(
        _tc2_body,
        grid=(GRID,),
        in_specs=[
            pl.BlockSpec((NC, BLK, D_HID), lambda i: (0, i, 0)),
            pl.BlockSpec((BLK, D_HID), lambda i: (i, 0)),
            pl.BlockSpec((NC, BLK), lambda i: (0, i)),
            pl.BlockSpec((D_HID, D_HID), lambda i: (0, 0)),
            pl.BlockSpec((1, D_HID), lambda i: (0, 0)),
        ],
        out_specs=pl.BlockSpec((BLK, D_HID), lambda i: (i, 0)),
        out_shape=jax.ShapeDtypeStruct((NP, D_HID), jnp.float32),
    )(S, hp, degp, W2, b1)


def _tc3(S, hp, degp, b2):
    return pl.pallas_call(
        _tc3_body,
        grid=(GRID,),
        in_specs=[
            pl.BlockSpec((NC, BLK, D_HID), lambda i: (0, i, 0)),
            pl.BlockSpec((BLK, D_HID), lambda i: (i, 0)),
            pl.BlockSpec((NC, BLK), lambda i: (0, i)),
            pl.BlockSpec((1, D_HID), lambda i: (0, 0)),
        ],
        out_specs=pl.BlockSpec((BLK, D_HID), lambda i: (i, 0)),
        out_shape=jax.ShapeDtypeStruct((NP, D_HID), jnp.float32),
    )(S, hp, degp, b2)


def kernel(x, W1, b1, W2, b2, edge_index):
    src = edge_index[0].astype(jnp.int32)
    dst = edge_index[1].astype(jnp.int32)
    pad = E_PAD - N_EDGES
    src_p = jnp.concatenate([src, jnp.zeros((pad,), jnp.int32)])
    dst_p = jnp.concatenate([dst, jnp.full((pad,), DUMMY, jnp.int32)])
    x_p = jnp.pad(x, ((0, NP - N_NODES), (0, 0)))
    b1r = b1.reshape(1, D_HID)
    b2r = b2.reshape(1, D_HID)

    src_q = src_p.reshape(E_PAD // QE, QE)
    dst_q = dst_p.reshape(E_PAD // QE, QE)

    degp = _sc_degree(dst_p)
    h1p = _tc1(x_p, W1, degp)
    S1 = _sc_aggregate(src_q, dst_q, h1p)
    h2p = _tc2(S1, h1p, degp, W2, b1r)
    S2 = _sc_aggregate(src_q, dst_q, h2p)
    out = _tc3(S2, h2p, degp, b2r)
    return out[:N_NODES]
